# Initial kernel scaffold; baseline (speedup 1.0000x reference)
import jax
import jax.numpy as jnp
from jax.experimental import pallas as pl


def kernel(x, Wg, W1, b1, W2, b2, Ws1, bs1, Ws2, bs2, alpha):
    raise NotImplementedError("write your pallas kernel here")


# speedup vs baseline: 1.0804x; 1.0804x over previous
import functools

import jax
import jax.numpy as jnp
from jax import lax
from jax.experimental import pallas as pl
from jax.experimental.pallas import tpu as pltpu
from jax.experimental.pallas import tpu_sc as plsc

T_TOK = 2048
C_DIM = 768
E_EXP = 8
H_DIM = 3072
S_SH = 2

BT = 256
P_SLOTS = T_TOK + E_EXP * BT
NB = P_SLOTS // BT
BTS = 256
NW = 32
VMEM_LIMIT = 128 * 1024 * 1024


def _router_body(t_ref, wg_ref, topi_ref):
    logits = jnp.dot(t_ref[...], wg_ref[...], preferred_element_type=jnp.float32)
    m = jnp.max(logits, axis=-1, keepdims=True)
    ii = lax.broadcasted_iota(jnp.int32, logits.shape, 1)
    topi_ref[...] = jnp.min(jnp.where(logits >= m, ii, E_EXP), axis=-1)


def _route(t, Wg):
    return pl.pallas_call(
        _router_body,
        out_shape=jax.ShapeDtypeStruct((T_TOK,), jnp.int32),
    )(t, Wg)


def _dispatch_meta(topi):
    order = jnp.argsort(topi).astype(jnp.int32)
    sorted_e = jnp.take(topi, order)
    counts = jnp.zeros((E_EXP,), jnp.int32).at[topi].add(1)
    padded = ((counts + BT - 1) // BT) * BT
    pad_end = jnp.cumsum(padded)
    pad_start = pad_end - padded
    off = jnp.cumsum(counts) - counts
    j = jnp.arange(T_TOK, dtype=jnp.int32)
    slot_j = jnp.take(pad_start, sorted_e) + (j - jnp.take(off, sorted_e))
    slot_token = jnp.zeros((P_SLOTS,), jnp.int32).at[slot_j].set(order)
    token_slot = jnp.zeros((T_TOK,), jnp.int32).at[order].set(slot_j)
    block_start = jnp.arange(NB, dtype=jnp.int32) * BT
    block_expert = jnp.minimum(
        jnp.searchsorted(pad_end, block_start, side="right"), E_EXP - 1
    ).astype(jnp.int32)
    return slot_token, token_slot, block_expert


def _make_sc_gather(n_out, d):
    b_per_w = n_out // NW
    mesh = plsc.VectorSubcoreMesh(core_axis_name="c", subcore_axis_name="s")

    @functools.partial(
        pl.kernel,
        mesh=mesh,
        out_type=jax.ShapeDtypeStruct((n_out, d), jnp.float32),
        scratch_types=[
            pltpu.VMEM((b_per_w,), jnp.int32),
            pltpu.VMEM((b_per_w, d), jnp.float32),
            pltpu.SemaphoreType.DMA,
        ],
    )
    def gk(table_hbm, idx_hbm, out_hbm, idx_v, rows_v, sem):
        wid = lax.axis_index("s") * 2 + lax.axis_index("c")
        base = wid * b_per_w
        pltpu.sync_copy(idx_hbm.at[pl.ds(base, b_per_w)], idx_v)
        pltpu.async_copy(table_hbm.at[idx_v], rows_v, sem).wait()
        pltpu.sync_copy(rows_v, out_hbm.at[pl.ds(base, b_per_w)])

    return gk


def _ffn_body(be_ref, xs_ref, w1_ref, b1_ref, w2_ref, b2_ref, ys_ref):
    h = jnp.dot(xs_ref[...], w1_ref[0], preferred_element_type=jnp.float32)
    h = jax.nn.gelu(h + b1_ref[0])
    y = jnp.dot(h, w2_ref[0], preferred_element_type=jnp.float32)
    ys_ref[...] = y + b2_ref[0]


def _routed_ffn(xs, W1, b1, W2, b2, block_expert):
    grid_spec = pltpu.PrefetchScalarGridSpec(
        num_scalar_prefetch=1,
        grid=(NB,),
        in_specs=[
            pl.BlockSpec((BT, C_DIM), lambda i, be: (i, 0)),
            pl.BlockSpec((1, C_DIM, H_DIM), lambda i, be: (be[i], 0, 0)),
            pl.BlockSpec((1, 1, H_DIM), lambda i, be: (be[i], 0, 0)),
            pl.BlockSpec((1, H_DIM, C_DIM), lambda i, be: (be[i], 0, 0)),
            pl.BlockSpec((1, 1, C_DIM), lambda i, be: (be[i], 0, 0)),
        ],
        out_specs=pl.BlockSpec((BT, C_DIM), lambda i, be: (i, 0)),
    )
    return pl.pallas_call(
        _ffn_body,
        grid_spec=grid_spec,
        out_shape=jax.ShapeDtypeStruct((P_SLOTS, C_DIM), jnp.float32),
        compiler_params=pltpu.CompilerParams(
            dimension_semantics=("arbitrary",),
            vmem_limit_bytes=VMEM_LIMIT,
        ),
    )(block_expert, xs, W1, b1.reshape(E_EXP, 1, H_DIM), W2,
      b2.reshape(E_EXP, 1, C_DIM))


def _shared_body(t_ref, rg_ref, ws1_ref, bs1_ref, ws2_ref, bs2_ref, alpha_ref,
                 out_ref):
    tb = t_ref[...]
    bs1 = bs1_ref[...]
    bs2 = bs2_ref[...]
    acc = rg_ref[...]
    for s in range(S_SH):
        h = jnp.dot(tb, ws1_ref[s], preferred_element_type=jnp.float32)
        h = jax.nn.gelu(h + bs1[s][None, :])
        y = jnp.dot(h, ws2_ref[s], preferred_element_type=jnp.float32)
        acc = acc + (1.0 / S_SH) * (y + bs2[s][None, :])
    out_ref[...] = tb + alpha_ref[0] * acc


def _shared_combine(t, rg, Ws1, bs1, Ws2, bs2, alpha):
    nblk = T_TOK // BTS
    return pl.pallas_call(
        _shared_body,
        grid=(nblk,),
        in_specs=[
            pl.BlockSpec((BTS, C_DIM), lambda i: (i, 0)),
            pl.BlockSpec((BTS, C_DIM), lambda i: (i, 0)),
            pl.BlockSpec((S_SH, C_DIM, H_DIM), lambda i: (0, 0, 0)),
            pl.BlockSpec((S_SH, H_DIM), lambda i: (0, 0)),
            pl.BlockSpec((S_SH, H_DIM, C_DIM), lambda i: (0, 0, 0)),
            pl.BlockSpec((S_SH, C_DIM), lambda i: (0, 0)),
            pl.BlockSpec(memory_space=pltpu.SMEM),
        ],
        out_specs=pl.BlockSpec((BTS, C_DIM), lambda i: (i, 0)),
        out_shape=jax.ShapeDtypeStruct((T_TOK, C_DIM), jnp.float32),
        compiler_params=pltpu.CompilerParams(
            dimension_semantics=("arbitrary",),
            vmem_limit_bytes=VMEM_LIMIT,
        ),
    )(t, rg, Ws1, bs1, Ws2, bs2, alpha)


def kernel(x, Wg, W1, b1, W2, b2, Ws1, bs1, Ws2, bs2, alpha):
    t = x.reshape(T_TOK, C_DIM)
    topi = _route(t, Wg)
    slot_token, token_slot, block_expert = _dispatch_meta(topi)
    xs = _make_sc_gather(P_SLOTS, C_DIM)(t, slot_token)
    ys = _routed_ffn(xs, W1, b1, W2, b2, block_expert)
    rg = _make_sc_gather(T_TOK, C_DIM)(ys, token_slot)
    out = _shared_combine(t, rg, Ws1, bs1, Ws2, bs2, alpha)
    return out.reshape(x.shape)


# speedup vs baseline: 1.1891x; 1.1005x over previous
import functools

import jax
import jax.numpy as jnp
from jax import lax
from jax.experimental import pallas as pl
from jax.experimental.pallas import tpu as pltpu
from jax.experimental.pallas import tpu_sc as plsc

T_TOK = 2048
C_DIM = 768
E_EXP = 8
H_DIM = 3072
S_SH = 2

BT = 128
P_SLOTS = T_TOK + E_EXP * BT
NB = P_SLOTS // BT
BTS = 256
NW = 32
VMEM_LIMIT = 128 * 1024 * 1024


def _router_body(t_ref, wg_ref, topi_ref):
    logits = jnp.dot(t_ref[...], wg_ref[...], preferred_element_type=jnp.float32)
    m = jnp.max(logits, axis=-1, keepdims=True)
    ii = lax.broadcasted_iota(jnp.int32, logits.shape, 1)
    topi_ref[...] = jnp.min(jnp.where(logits >= m, ii, E_EXP), axis=-1)


def _route(t, Wg):
    return pl.pallas_call(
        _router_body,
        out_shape=jax.ShapeDtypeStruct((T_TOK,), jnp.int32),
    )(t, Wg)


def _dispatch_meta(topi):
    order = jnp.argsort(topi).astype(jnp.int32)
    sorted_e = jnp.take(topi, order)
    counts = jnp.zeros((E_EXP,), jnp.int32).at[topi].add(1)
    padded = ((counts + BT - 1) // BT) * BT
    pad_end = jnp.cumsum(padded)
    pad_start = pad_end - padded
    off = jnp.cumsum(counts) - counts
    j = jnp.arange(T_TOK, dtype=jnp.int32)
    slot_j = jnp.take(pad_start, sorted_e) + (j - jnp.take(off, sorted_e))
    slot_token = jnp.zeros((P_SLOTS,), jnp.int32).at[slot_j].set(order)
    token_slot = jnp.zeros((T_TOK,), jnp.int32).at[order].set(slot_j)
    block_start = jnp.arange(NB, dtype=jnp.int32) * BT
    block_expert = jnp.minimum(
        jnp.searchsorted(pad_end, block_start, side="right"), E_EXP - 1
    ).astype(jnp.int32)
    return slot_token, token_slot, block_expert


def _make_sc_gather(n_out, d, nch):
    b_per_w = n_out // NW
    ch = b_per_w // nch
    mesh = plsc.VectorSubcoreMesh(core_axis_name="c", subcore_axis_name="s")

    @functools.partial(
        pl.kernel,
        mesh=mesh,
        out_type=jax.ShapeDtypeStruct((n_out, d), jnp.float32),
        scratch_types=[
            pltpu.VMEM((b_per_w,), jnp.int32),
            pltpu.VMEM((b_per_w, d), jnp.float32),
            pltpu.SemaphoreType.DMA,
        ],
    )
    def gk(table_hbm, idx_hbm, out_hbm, idx_v, rows_v, sem):
        wid = lax.axis_index("s") * 2 + lax.axis_index("c")
        base = wid * b_per_w
        pltpu.sync_copy(idx_hbm.at[pl.ds(base, b_per_w)], idx_v)
        copies = [
            pltpu.make_async_copy(
                table_hbm.at[idx_v.at[pl.ds(k * ch, ch)]],
                rows_v.at[pl.ds(k * ch, ch)],
                sem,
            )
            for k in range(nch)
        ]
        for c in copies:
            c.start()
        for c in copies:
            c.wait()
        pltpu.sync_copy(rows_v, out_hbm.at[pl.ds(base, b_per_w)])

    return gk


def _ffn_body(be_ref, xs_ref, w1_ref, b1_ref, w2_ref, b2_ref, ys_ref):
    h = jnp.dot(xs_ref[...], w1_ref[0], preferred_element_type=jnp.float32)
    h = jax.nn.gelu(h + b1_ref[0])
    y = jnp.dot(h, w2_ref[0], preferred_element_type=jnp.float32)
    ys_ref[...] = y + b2_ref[0]


def _routed_ffn(xs, W1, b1, W2, b2, block_expert):
    grid_spec = pltpu.PrefetchScalarGridSpec(
        num_scalar_prefetch=1,
        grid=(NB,),
        in_specs=[
            pl.BlockSpec((BT, C_DIM), lambda i, be: (i, 0)),
            pl.BlockSpec((1, C_DIM, H_DIM), lambda i, be: (be[i], 0, 0)),
            pl.BlockSpec((1, 1, H_DIM), lambda i, be: (be[i], 0, 0)),
            pl.BlockSpec((1, H_DIM, C_DIM), lambda i, be: (be[i], 0, 0)),
            pl.BlockSpec((1, 1, C_DIM), lambda i, be: (be[i], 0, 0)),
        ],
        out_specs=pl.BlockSpec((BT, C_DIM), lambda i, be: (i, 0)),
    )
    return pl.pallas_call(
        _ffn_body,
        grid_spec=grid_spec,
        out_shape=jax.ShapeDtypeStruct((P_SLOTS, C_DIM), jnp.float32),
        compiler_params=pltpu.CompilerParams(
            dimension_semantics=("arbitrary",),
            vmem_limit_bytes=VMEM_LIMIT,
        ),
    )(block_expert, xs, W1, b1.reshape(E_EXP, 1, H_DIM), W2,
      b2.reshape(E_EXP, 1, C_DIM))


def _shared_body(t_ref, ws1_ref, bs1_ref, ws2_ref, bs2_ref, alpha_ref, out_ref):
    tb = t_ref[...]
    bs1 = bs1_ref[...]
    bs2 = bs2_ref[...]
    acc = jnp.zeros_like(tb)
    for s in range(S_SH):
        h = jnp.dot(tb, ws1_ref[s], preferred_element_type=jnp.float32)
        h = jax.nn.gelu(h + bs1[s][None, :])
        y = jnp.dot(h, ws2_ref[s], preferred_element_type=jnp.float32)
        acc = acc + (1.0 / S_SH) * (y + bs2[s][None, :])
    out_ref[...] = tb + alpha_ref[0] * acc


def _shared_ffn(t, Ws1, bs1, Ws2, bs2, alpha):
    nblk = T_TOK // BTS
    return pl.pallas_call(
        _shared_body,
        grid=(nblk,),
        in_specs=[
            pl.BlockSpec((BTS, C_DIM), lambda i: (i, 0)),
            pl.BlockSpec((S_SH, C_DIM, H_DIM), lambda i: (0, 0, 0)),
            pl.BlockSpec((S_SH, H_DIM), lambda i: (0, 0)),
            pl.BlockSpec((S_SH, H_DIM, C_DIM), lambda i: (0, 0, 0)),
            pl.BlockSpec((S_SH, C_DIM), lambda i: (0, 0)),
            pl.BlockSpec(memory_space=pltpu.SMEM),
        ],
        out_specs=pl.BlockSpec((BTS, C_DIM), lambda i: (i, 0)),
        out_shape=jax.ShapeDtypeStruct((T_TOK, C_DIM), jnp.float32),
        compiler_params=pltpu.CompilerParams(
            dimension_semantics=("arbitrary",),
            vmem_limit_bytes=VMEM_LIMIT,
        ),
    )(t, Ws1, bs1, Ws2, bs2, alpha)


def _combine_body(sh_ref, rg_ref, alpha_ref, out_ref):
    out_ref[...] = sh_ref[...] + alpha_ref[0] * rg_ref[...]


def _combine(sh, rg, alpha):
    return pl.pallas_call(
        _combine_body,
        in_specs=[
            pl.BlockSpec((T_TOK, C_DIM), lambda: (0, 0)),
            pl.BlockSpec((T_TOK, C_DIM), lambda: (0, 0)),
            pl.BlockSpec(memory_space=pltpu.SMEM),
        ],
        out_specs=pl.BlockSpec((T_TOK, C_DIM), lambda: (0, 0)),
        out_shape=jax.ShapeDtypeStruct((T_TOK, C_DIM), jnp.float32),
    )(sh, rg, alpha)


def kernel(x, Wg, W1, b1, W2, b2, Ws1, bs1, Ws2, bs2, alpha):
    t = x.reshape(T_TOK, C_DIM)
    topi = _route(t, Wg)
    slot_token, token_slot, block_expert = _dispatch_meta(topi)
    xs = _make_sc_gather(P_SLOTS, C_DIM, nch=6)(t, slot_token)
    sh = _shared_ffn(t, Ws1, bs1, Ws2, bs2, alpha)
    ys = _routed_ffn(xs, W1, b1, W2, b2, block_expert)
    rg = _make_sc_gather(T_TOK, C_DIM, nch=4)(ys, token_slot)
    out = _combine(sh, rg, alpha)
    return out.reshape(x.shape)


# speedup vs baseline: 1.4265x; 1.1997x over previous
import functools

import jax
import jax.numpy as jnp
from jax import lax
from jax.experimental import pallas as pl
from jax.experimental.pallas import tpu as pltpu
from jax.experimental.pallas import tpu_sc as plsc

T_TOK = 2048
C_DIM = 768
E_EXP = 8
H_DIM = 3072
S_SH = 2

BT = 128
P_SLOTS = T_TOK + E_EXP * BT
NB = P_SLOTS // BT
BTS = 256
NW = 32
VMEM_LIMIT = 128 * 1024 * 1024


def _router_body(t_ref, wg_ref, topi_ref):
    logits = jnp.dot(t_ref[...], wg_ref[...], preferred_element_type=jnp.float32)
    m = jnp.max(logits, axis=-1, keepdims=True)
    ii = lax.broadcasted_iota(jnp.int32, logits.shape, 1)
    topi_ref[...] = jnp.min(jnp.where(logits >= m, ii, E_EXP), axis=-1)


def _route(t, Wg):
    return pl.pallas_call(
        _router_body,
        out_shape=jax.ShapeDtypeStruct((T_TOK,), jnp.int32),
    )(t, Wg)


def _dispatch_meta(topi):
    order = jnp.argsort(topi).astype(jnp.int32)
    sorted_e = jnp.take(topi, order)
    counts = jnp.zeros((E_EXP,), jnp.int32).at[topi].add(1)
    padded = ((counts + BT - 1) // BT) * BT
    pad_end = jnp.cumsum(padded)
    pad_start = pad_end - padded
    off = jnp.cumsum(counts) - counts
    j = jnp.arange(T_TOK, dtype=jnp.int32)
    slot_j = jnp.take(pad_start, sorted_e) + (j - jnp.take(off, sorted_e))
    filler = jnp.arange(P_SLOTS, dtype=jnp.int32) % T_TOK
    slot_token = filler.at[slot_j].set(order)
    token_slot = jnp.zeros((T_TOK,), jnp.int32).at[order].set(slot_j)
    block_start = jnp.arange(NB, dtype=jnp.int32) * BT
    block_expert = jnp.minimum(
        jnp.searchsorted(pad_end, block_start, side="right"), E_EXP - 1
    ).astype(jnp.int32)
    return slot_token, token_slot, block_expert


def _make_sc_gather(n_out, d, nch):
    b_per_w = n_out // NW
    ch = b_per_w // nch
    mesh = plsc.VectorSubcoreMesh(core_axis_name="c", subcore_axis_name="s")

    @functools.partial(
        pl.kernel,
        mesh=mesh,
        out_type=jax.ShapeDtypeStruct((n_out, d), jnp.float32),
        scratch_types=[
            pltpu.VMEM((b_per_w,), jnp.int32),
            pltpu.VMEM((b_per_w, d), jnp.float32),
            pltpu.SemaphoreType.DMA,
        ],
    )
    def gk(table_hbm, idx_hbm, out_hbm, idx_v, rows_v, sem):
        wid = lax.axis_index("s") * 2 + lax.axis_index("c")
        base = wid * b_per_w
        pltpu.sync_copy(idx_hbm.at[pl.ds(base, b_per_w)], idx_v)
        copies = [
            pltpu.make_async_copy(
                table_hbm.at[idx_v.at[pl.ds(k * ch, ch)]],
                rows_v.at[pl.ds(k * ch, ch)],
                sem,
            )
            for k in range(nch)
        ]
        for c in copies:
            c.start()
        for c in copies:
            c.wait()
        pltpu.sync_copy(rows_v, out_hbm.at[pl.ds(base, b_per_w)])

    return gk


def _ffn_body(be_ref, xs_ref, w1_ref, b1_ref, w2_ref, b2_ref, ys_ref):
    h = jnp.dot(xs_ref[...], w1_ref[0], preferred_element_type=jnp.float32)
    h = jax.nn.gelu(h + b1_ref[0])
    y = jnp.dot(h, w2_ref[0], preferred_element_type=jnp.float32)
    ys_ref[...] = y + b2_ref[0]


def _routed_ffn(xs, W1, b1, W2, b2, block_expert):
    grid_spec = pltpu.PrefetchScalarGridSpec(
        num_scalar_prefetch=1,
        grid=(NB,),
        in_specs=[
            pl.BlockSpec((BT, C_DIM), lambda i, be: (i, 0)),
            pl.BlockSpec((1, C_DIM, H_DIM), lambda i, be: (be[i], 0, 0)),
            pl.BlockSpec((1, 1, H_DIM), lambda i, be: (be[i], 0, 0)),
            pl.BlockSpec((1, H_DIM, C_DIM), lambda i, be: (be[i], 0, 0)),
            pl.BlockSpec((1, 1, C_DIM), lambda i, be: (be[i], 0, 0)),
        ],
        out_specs=pl.BlockSpec((BT, C_DIM), lambda i, be: (i, 0)),
    )
    return pl.pallas_call(
        _ffn_body,
        grid_spec=grid_spec,
        out_shape=jax.ShapeDtypeStruct((P_SLOTS, C_DIM), jnp.float32),
        compiler_params=pltpu.CompilerParams(
            dimension_semantics=("arbitrary",),
            vmem_limit_bytes=VMEM_LIMIT,
        ),
    )(block_expert, xs, W1, b1.reshape(E_EXP, 1, H_DIM), W2,
      b2.reshape(E_EXP, 1, C_DIM))


def _shared_body(t_ref, ws1_ref, bs1_ref, ws2_ref, bs2_ref, alpha_ref, out_ref):
    tb = t_ref[...]
    bs1 = bs1_ref[...]
    bs2 = bs2_ref[...]
    acc = jnp.zeros_like(tb)
    for s in range(S_SH):
        h = jnp.dot(tb, ws1_ref[s], preferred_element_type=jnp.float32)
        h = jax.nn.gelu(h + bs1[s][None, :])
        y = jnp.dot(h, ws2_ref[s], preferred_element_type=jnp.float32)
        acc = acc + (1.0 / S_SH) * (y + bs2[s][None, :])
    out_ref[...] = tb + alpha_ref[0] * acc


def _shared_ffn(t, Ws1, bs1, Ws2, bs2, alpha):
    nblk = T_TOK // BTS
    return pl.pallas_call(
        _shared_body,
        grid=(nblk,),
        in_specs=[
            pl.BlockSpec((BTS, C_DIM), lambda i: (i, 0)),
            pl.BlockSpec((S_SH, C_DIM, H_DIM), lambda i: (0, 0, 0)),
            pl.BlockSpec((S_SH, H_DIM), lambda i: (0, 0)),
            pl.BlockSpec((S_SH, H_DIM, C_DIM), lambda i: (0, 0, 0)),
            pl.BlockSpec((S_SH, C_DIM), lambda i: (0, 0)),
            pl.BlockSpec(memory_space=pltpu.SMEM),
        ],
        out_specs=pl.BlockSpec((BTS, C_DIM), lambda i: (i, 0)),
        out_shape=jax.ShapeDtypeStruct((T_TOK, C_DIM), jnp.float32),
        compiler_params=pltpu.CompilerParams(
            dimension_semantics=("arbitrary",),
            vmem_limit_bytes=VMEM_LIMIT,
        ),
    )(t, Ws1, bs1, Ws2, bs2, alpha)


def _combine_body(sh_ref, rg_ref, alpha_ref, out_ref):
    out_ref[...] = sh_ref[...] + alpha_ref[0] * rg_ref[...]


def _combine(sh, rg, alpha):
    return pl.pallas_call(
        _combine_body,
        in_specs=[
            pl.BlockSpec((T_TOK, C_DIM), lambda: (0, 0)),
            pl.BlockSpec((T_TOK, C_DIM), lambda: (0, 0)),
            pl.BlockSpec(memory_space=pltpu.SMEM),
        ],
        out_specs=pl.BlockSpec((T_TOK, C_DIM), lambda: (0, 0)),
        out_shape=jax.ShapeDtypeStruct((T_TOK, C_DIM), jnp.float32),
    )(sh, rg, alpha)


def kernel(x, Wg, W1, b1, W2, b2, Ws1, bs1, Ws2, bs2, alpha):
    t = x.reshape(T_TOK, C_DIM)
    topi = _route(t, Wg)
    slot_token, token_slot, block_expert = _dispatch_meta(topi)
    xs = _make_sc_gather(P_SLOTS, C_DIM, nch=12)(t, slot_token)
    sh = _shared_ffn(t, Ws1, bs1, Ws2, bs2, alpha)
    ys = _routed_ffn(xs, W1, b1, W2, b2, block_expert)
    rg = _make_sc_gather(T_TOK, C_DIM, nch=4)(ys, token_slot)
    out = _combine(sh, rg, alpha)
    return out.reshape(x.shape)
